# baseline (speedup 1.0000x reference)
import functools

import jax
import jax.numpy as jnp
from jax import lax
from jax.experimental import pallas as pl
from jax.experimental.pallas import tpu as pltpu
from jax.experimental.pallas import tpu_sc as plsc

_NC, _NS = 2, 16
_W = 128


def _seg_sum_partials(tables, src_w, dst_w, zeros, acc_rows, dc, wpt):
    nt = len(tables)
    rpt = acc_rows // _NS
    mesh = plsc.VectorSubcoreMesh(core_axis_name="c", subcore_axis_name="s")

    @functools.partial(
        pl.kernel,
        out_type=jax.ShapeDtypeStruct((nt, _NC, acc_rows, dc), jnp.float32),
        mesh=mesh,
        scratch_types=[
            pltpu.VMEM_SHARED((acc_rows, dc), jnp.float32),
            pltpu.VMEM((wpt // 2, _W), jnp.int32),
            pltpu.VMEM((wpt // 2, _W), jnp.int32),
            pltpu.VMEM((2, _W, dc), jnp.float32),
            pltpu.SemaphoreType.DMA,
            pltpu.SemaphoreType.DMA,
        ],
    )
    def seg(*refs):
        table_hs = refs[:nt]
        src_h, dst_h, zero_h, out_h = refs[nt:nt + 4]
        acc, srcv, dstv, rows, gs0, gs1 = refs[nt + 4:]
        cid = lax.axis_index("c")
        sid = lax.axis_index("s")
        tid = cid * _NS + sid
        hw = wpt // 2
        for c in range(nt):
            table_h = table_hs[c]
            pltpu.sync_copy(zero_h.at[pl.ds(sid * rpt, rpt)],
                            acc.at[pl.ds(sid * rpt, rpt)])
            plsc.subcore_barrier()

            for h in range(2):
                pltpu.sync_copy(src_h.at[pl.ds(tid * wpt + h * hw, hw)], srcv)
                pltpu.sync_copy(dst_h.at[pl.ds(tid * wpt + h * hw, hw)], dstv)
                pltpu.async_copy(table_h.at[srcv.at[0]], rows.at[0], gs0)

                def body(i, carry, table_h=table_h):
                    j0 = 2 * i
                    pltpu.async_copy(table_h.at[srcv.at[j0 + 1]], rows.at[1],
                                     gs1)
                    pltpu.make_async_copy(table_h.at[srcv.at[j0]], rows.at[0],
                                          gs0).wait()
                    pltpu.sync_copy(rows.at[0], acc.at[dstv.at[j0]], add=True)

                    @pl.when(j0 + 2 < hw)
                    def _():
                        pltpu.async_copy(table_h.at[srcv.at[j0 + 2]],
                                         rows.at[0], gs0)

                    pltpu.make_async_copy(table_h.at[srcv.at[j0 + 1]],
                                          rows.at[1], gs1).wait()
                    pltpu.sync_copy(rows.at[1], acc.at[dstv.at[j0 + 1]],
                                    add=True)
                    return carry

                lax.fori_loop(0, hw // 2, body, 0)
            plsc.subcore_barrier()
            pltpu.sync_copy(acc.at[pl.ds(sid * rpt, rpt)],
                            out_h.at[c, cid, pl.ds(sid * rpt, rpt)])

    return seg(*tables, src_w, dst_w, zeros)


def _relu(v):
    return jnp.maximum(v, 0.0)


def _stage_a(xp, ep, Wg, bg, Wc, bc, Wfc, bfc, color, np_, bn):
    def body(x_r, e_r, wg_r, bg_r, wc_r, bc_r, wfc_r, bfc_r, f_r, oth_r):
        xh = _relu(lax.dot_general(x_r[...], wg_r[...],
                                   (((0,), (1,)), ((), ()))) + bg_r[...])
        eh = _relu(lax.dot_general(e_r[...], wc_r[...],
                                   (((0,), (1,)), ((), ()))) + bc_r[...])
        feat, t = (eh, xh) if color else (xh, eh)
        f_r[...] = jnp.concatenate(
            [feat, jnp.zeros(feat.shape, jnp.float32)], axis=1)
        oth_r[...] = _relu(lax.dot_general(t, wfc_r[...],
                                           (((1,), (1,)), ((), ()))) + bfc_r[...])

    return pl.pallas_call(
        body,
        grid=(np_ // bn,),
        in_specs=[
            pl.BlockSpec((3, bn), lambda i: (0, i)),
            pl.BlockSpec((32, bn), lambda i: (0, i)),
            pl.BlockSpec((64, 3), lambda i: (0, 0)),
            pl.BlockSpec((1, 64), lambda i: (0, 0)),
            pl.BlockSpec((64, 32), lambda i: (0, 0)),
            pl.BlockSpec((1, 64), lambda i: (0, 0)),
            pl.BlockSpec((256, 64), lambda i: (0, 0)),
            pl.BlockSpec((1, 256), lambda i: (0, 0)),
        ],
        out_specs=[
            pl.BlockSpec((bn, 128), lambda i: (i, 0)),
            pl.BlockSpec((bn, 256), lambda i: (i, 0)),
        ],
        out_shape=[
            jax.ShapeDtypeStruct((np_, 128), jnp.float32),
            jax.ShapeDtypeStruct((np_, 256), jnp.float32),
        ],
    )(xp, ep, Wg, bg, Wc, bc, Wfc, bfc)


def _gin_mlp(f, a0, a1, Wa, ba, Wb, bb, bbcol, relu_out, np_, bn):
    cin, chid = Wa.shape[1], Wa.shape[0]
    cout = Wb.shape[0]
    ca = f.shape[1]

    def body(f_r, a0_r, a1_r, wa_r, ba_r, wb_r, bb_r, bbc_r, o_r, om_r):
        h = (f_r[...] + a0_r[...] + a1_r[...])[:, :cin]
        z = _relu(lax.dot_general(h, wa_r[...], (((1,), (1,)), ((), ()))) + ba_r[...])
        o = lax.dot_general(z, wb_r[...], (((1,), (1,)), ((), ()))) + bb_r[...]
        om = lax.dot_general(wb_r[...], z, (((1,), (1,)), ((), ()))) + bbc_r[...]
        if relu_out:
            o, om = _relu(o), _relu(om)
        o_r[...] = o
        om_r[...] = om

    return pl.pallas_call(
        body,
        grid=(np_ // bn,),
        in_specs=[
            pl.BlockSpec((bn, ca), lambda i: (i, 0)),
            pl.BlockSpec((bn, ca), lambda i: (i, 0)),
            pl.BlockSpec((bn, ca), lambda i: (i, 0)),
            pl.BlockSpec((chid, cin), lambda i: (0, 0)),
            pl.BlockSpec((1, chid), lambda i: (0, 0)),
            pl.BlockSpec((cout, chid), lambda i: (0, 0)),
            pl.BlockSpec((1, cout), lambda i: (0, 0)),
            pl.BlockSpec((cout, 1), lambda i: (0, 0)),
        ],
        out_specs=[
            pl.BlockSpec((bn, cout), lambda i: (i, 0)),
            pl.BlockSpec((cout, bn), lambda i: (0, i)),
        ],
        out_shape=[
            jax.ShapeDtypeStruct((np_, cout), jnp.float32),
            jax.ShapeDtypeStruct((cout, np_), jnp.float32),
        ],
    )(f, a0, a1, Wa, ba, Wb, bb, bbcol)


def _stage_d(chunks, parts, W3a, b3a, W3b, b3b, b3bcol, np_, bn):
    def body(c0, c1, c2, c3, p00, p01, p10, p11, p20, p21, p30, p31,
             wa_r, ba_r, wb_r, bbc_r, om_r):
        hs = [c0[...] + p00[...] + p01[...],
              c1[...] + p10[...] + p11[...],
              c2[...] + p20[...] + p21[...],
              c3[...] + p30[...] + p31[...]]
        wa = wa_r[...]
        acc = lax.dot_general(hs[0], wa[:, 0:128], (((1,), (1,)), ((), ())))
        for k in range(1, 4):
            acc = acc + lax.dot_general(hs[k], wa[:, 128 * k:128 * (k + 1)],
                                        (((1,), (1,)), ((), ())))
        z = _relu(acc + ba_r[...])
        om_r[...] = lax.dot_general(wb_r[...], z, (((1,), (1,)), ((), ()))) + bbc_r[...]

    blk = lambda shape, imap: pl.BlockSpec(shape, imap)
    node_in = [blk((bn, 128), lambda i: (i, 0)) for _ in range(12)]
    return pl.pallas_call(
        body,
        grid=(np_ // bn,),
        in_specs=node_in + [
            blk((512, 512), lambda i: (0, 0)),
            blk((1, 512), lambda i: (0, 0)),
            blk((1024, 512), lambda i: (0, 0)),
            blk((1024, 1), lambda i: (0, 0)),
        ],
        out_specs=[blk((1024, bn), lambda i: (0, i))],
        out_shape=[jax.ShapeDtypeStruct((1024, np_), jnp.float32)],
    )(*chunks, *parts, W3a, b3a, W3b, b3bcol)[0]


def kernel(x, emb, graph_data, e, Wg, bg, Wc, bc, W1a, b1a, W1b, b1b,
           W2a, b2a, W2b, b2b, W3a, b3a, W3b, b3b, Wfc, bfc):
    n = x.shape[2]
    ne = graph_data.shape[1]
    np_ = ((n + 1023) // 1024) * 1024
    npa = np_ + 128
    bn = 2048
    nwin = -(-(ne // _W) // (_NC * _NS * 8)) * (_NC * _NS * 8)
    wpt = nwin // (_NC * _NS)

    color = (e == 'color')

    xp = jnp.pad(x[0], ((0, 0), (0, np_ - n)))
    ep = jnp.pad(emb[0], ((0, 0), (0, np_ - n)))
    src, dst = graph_data[0], graph_data[1]
    pad_e = nwin * _W - ne
    ar = jnp.arange(pad_e, dtype=jnp.int32)
    src_w = jnp.concatenate([src, (ar * 37) % jnp.int32(n)]).reshape(nwin, _W)
    dst_w = jnp.concatenate([dst, np_ + (ar % 16)]).reshape(nwin, _W)
    z128 = jnp.zeros((npa, 128), jnp.float32)
    row = lambda b: b.reshape(1, -1)
    col = lambda b: b.reshape(-1, 1)

    f, oth = _stage_a(xp, ep, Wg, row(bg), Wc, row(bc), Wfc, row(bfc),
                      color, np_, bn)

    a1 = _seg_sum_partials([f], src_w, dst_w, z128, npa, 128, wpt)[0]
    f1, f1m = _gin_mlp(f, a1[0, :np_], a1[1, :np_], W1a, row(b1a),
                       W1b, row(b1b), col(b1b), True, np_, bn)

    a2 = _seg_sum_partials([f1], src_w, dst_w, z128, npa, 128, wpt)[0]
    f2, f2m = _gin_mlp(f1, a2[0, :np_], a2[1, :np_], W2a, row(b2a),
                       W2b, row(b2b), col(b2b), True, np_, bn)

    chunks = [f2[:, :128], f2[:, 128:], oth[:, :128], oth[:, 128:]]
    a3 = _seg_sum_partials(chunks, src_w, dst_w, z128, npa, 128, wpt)
    parts = [a3[c, p, :np_] for c in range(4) for p in range(2)]
    f3m = _stage_d(chunks, parts, W3a, row(b3a), W3b, row(b3b), col(b3b),
                   np_, bn)

    out = jnp.concatenate([f1m, f2m, f3m], axis=0)[:, :n]
    return out[None]


# speedup vs baseline: 1.0618x; 1.0618x over previous
import functools

import jax
import jax.numpy as jnp
from jax import lax
from jax.experimental import pallas as pl
from jax.experimental.pallas import tpu as pltpu
from jax.experimental.pallas import tpu_sc as plsc

_NC, _NS = 2, 16
_W = 128


def _seg_sum_partials(tables, src_w, dst_w, zeros, acc_rows, dc, wpt):
    nt = len(tables)
    rpt = acc_rows // _NS
    mesh = plsc.VectorSubcoreMesh(core_axis_name="c", subcore_axis_name="s")

    @functools.partial(
        pl.kernel,
        out_type=jax.ShapeDtypeStruct((nt, _NC, acc_rows, dc), jnp.float32),
        mesh=mesh,
        scratch_types=[
            pltpu.VMEM_SHARED((acc_rows, dc), jnp.float32),
            pltpu.VMEM((wpt // 2, _W), jnp.int32),
            pltpu.VMEM((wpt // 2, _W), jnp.int32),
            pltpu.VMEM((2, _W, dc), jnp.float32),
            pltpu.SemaphoreType.DMA,
            pltpu.SemaphoreType.DMA,
        ],
    )
    def seg(*refs):
        table_hs = refs[:nt]
        src_h, dst_h, zero_h, out_h = refs[nt:nt + 4]
        acc, srcv, dstv, rows, gs0, gs1 = refs[nt + 4:]
        cid = lax.axis_index("c")
        sid = lax.axis_index("s")
        tid = cid * _NS + sid
        hw = wpt // 2
        for c in range(nt):
            table_h = table_hs[c]
            pltpu.sync_copy(zero_h.at[pl.ds(sid * rpt, rpt)],
                            acc.at[pl.ds(sid * rpt, rpt)])
            plsc.subcore_barrier()

            for h in range(2):
                pltpu.sync_copy(src_h.at[pl.ds(tid * wpt + h * hw, hw)], srcv)
                pltpu.sync_copy(dst_h.at[pl.ds(tid * wpt + h * hw, hw)], dstv)
                pltpu.async_copy(table_h.at[srcv.at[0]], rows.at[0], gs0)

                def body(i, carry, table_h=table_h):
                    j0 = 2 * i
                    pltpu.async_copy(table_h.at[srcv.at[j0 + 1]], rows.at[1],
                                     gs1)
                    pltpu.make_async_copy(table_h.at[srcv.at[j0]], rows.at[0],
                                          gs0).wait()
                    pltpu.sync_copy(rows.at[0], acc.at[dstv.at[j0]], add=True)

                    @pl.when(j0 + 2 < hw)
                    def _():
                        pltpu.async_copy(table_h.at[srcv.at[j0 + 2]],
                                         rows.at[0], gs0)

                    pltpu.make_async_copy(table_h.at[srcv.at[j0 + 1]],
                                          rows.at[1], gs1).wait()
                    pltpu.sync_copy(rows.at[1], acc.at[dstv.at[j0 + 1]],
                                    add=True)
                    return carry

                lax.fori_loop(0, hw // 2, body, 0)
            plsc.subcore_barrier()
            pltpu.sync_copy(acc.at[pl.ds(sid * rpt, rpt)],
                            out_h.at[c, cid, pl.ds(sid * rpt, rpt)])

    return seg(*tables, src_w, dst_w, zeros)


def _relu(v):
    return jnp.maximum(v, 0.0)


def _stage_a(xp, ep, Wg, bg, Wc, bc, Wfc, bfc, color, np_, bn):
    def body(x_r, e_r, wg_r, bg_r, wc_r, bc_r, wfc_r, bfc_r, f_r,
             oth0_r, oth1_r):
        xh = _relu(lax.dot_general(x_r[...], wg_r[...],
                                   (((0,), (1,)), ((), ()))) + bg_r[...])
        eh = _relu(lax.dot_general(e_r[...], wc_r[...],
                                   (((0,), (1,)), ((), ()))) + bc_r[...])
        feat, t = (eh, xh) if color else (xh, eh)
        f_r[...] = jnp.concatenate(
            [feat, jnp.zeros(feat.shape, jnp.float32)], axis=1)
        oth = _relu(lax.dot_general(t, wfc_r[...],
                                    (((1,), (1,)), ((), ()))) + bfc_r[...])
        oth0_r[...] = oth[:, :128]
        oth1_r[...] = oth[:, 128:]

    return pl.pallas_call(
        body,
        grid=(np_ // bn,),
        in_specs=[
            pl.BlockSpec((3, bn), lambda i: (0, i)),
            pl.BlockSpec((32, bn), lambda i: (0, i)),
            pl.BlockSpec((64, 3), lambda i: (0, 0)),
            pl.BlockSpec((1, 64), lambda i: (0, 0)),
            pl.BlockSpec((64, 32), lambda i: (0, 0)),
            pl.BlockSpec((1, 64), lambda i: (0, 0)),
            pl.BlockSpec((256, 64), lambda i: (0, 0)),
            pl.BlockSpec((1, 256), lambda i: (0, 0)),
        ],
        out_specs=[
            pl.BlockSpec((bn, 128), lambda i: (i, 0)),
            pl.BlockSpec((bn, 128), lambda i: (i, 0)),
            pl.BlockSpec((bn, 128), lambda i: (i, 0)),
        ],
        out_shape=[
            jax.ShapeDtypeStruct((np_, 128), jnp.float32),
            jax.ShapeDtypeStruct((np_, 128), jnp.float32),
            jax.ShapeDtypeStruct((np_, 128), jnp.float32),
        ],
    )(xp, ep, Wg, bg, Wc, bc, Wfc, bfc)


def _gin_mlp(f, agg, Wa, ba, Wb, bb, bbcol, relu_out, np_, bn):
    cin, chid = Wa.shape[1], Wa.shape[0]
    cout = Wb.shape[0]
    ca = f.shape[1]
    nsp = cout // 128

    def body(f_r, a0_r, a1_r, wa_r, ba_r, wb_r, bb_r, bbc_r, *outs):
        h = (f_r[...] + a0_r[...][0] + a1_r[...][0])[:, :cin]
        z = _relu(lax.dot_general(h, wa_r[...], (((1,), (1,)), ((), ()))) + ba_r[...])
        o = lax.dot_general(z, wb_r[...], (((1,), (1,)), ((), ()))) + bb_r[...]
        om = lax.dot_general(wb_r[...], z, (((1,), (1,)), ((), ()))) + bbc_r[...]
        if relu_out:
            o, om = _relu(o), _relu(om)
        for k in range(nsp):
            outs[k][...] = o[:, 128 * k:128 * (k + 1)]
        outs[nsp][...] = om

    res = pl.pallas_call(
        body,
        grid=(np_ // bn,),
        in_specs=[
            pl.BlockSpec((bn, ca), lambda i: (i, 0)),
            pl.BlockSpec((1, bn, ca), lambda i: (0, i, 0)),
            pl.BlockSpec((1, bn, ca), lambda i: (1, i, 0)),
            pl.BlockSpec((chid, cin), lambda i: (0, 0)),
            pl.BlockSpec((1, chid), lambda i: (0, 0)),
            pl.BlockSpec((cout, chid), lambda i: (0, 0)),
            pl.BlockSpec((1, cout), lambda i: (0, 0)),
            pl.BlockSpec((cout, 1), lambda i: (0, 0)),
        ],
        out_specs=[pl.BlockSpec((bn, 128), lambda i: (i, 0))
                   for _ in range(nsp)] +
                  [pl.BlockSpec((cout, bn), lambda i: (0, i))],
        out_shape=[jax.ShapeDtypeStruct((np_, 128), jnp.float32)
                   for _ in range(nsp)] +
                  [jax.ShapeDtypeStruct((cout, np_), jnp.float32)],
    )(f, agg, agg, Wa, ba, Wb, bb, bbcol)
    return res[:nsp], res[nsp]


def _stage_d(chunks, a3, W3a, b3a, W3b, b3b, b3bcol, np_, bn):
    def body(c0, c1, c2, c3, p00, p01, p10, p11, p20, p21, p30, p31,
             wa_r, ba_r, wb_r, bbc_r, om_r):
        ps = [p00, p01, p10, p11, p20, p21, p30, p31]
        cs = [c0, c1, c2, c3]
        hs = [cs[k][...] + ps[2 * k][...][0, 0] + ps[2 * k + 1][...][0, 0]
              for k in range(4)]
        wa = wa_r[...]
        acc = lax.dot_general(hs[0], wa[:, 0:128], (((1,), (1,)), ((), ())))
        for k in range(1, 4):
            acc = acc + lax.dot_general(hs[k], wa[:, 128 * k:128 * (k + 1)],
                                        (((1,), (1,)), ((), ())))
        z = _relu(acc + ba_r[...])
        om_r[...] = lax.dot_general(wb_r[...], z, (((1,), (1,)), ((), ()))) + bbc_r[...]

    blk = lambda shape, imap: pl.BlockSpec(shape, imap)
    node_in = [blk((bn, 128), lambda i: (i, 0)) for _ in range(4)]
    part_in = [blk((1, 1, bn, 128),
                   functools.partial(lambda c, p, i: (c, p, i, 0), c, p))
               for c in range(4) for p in range(2)]
    return pl.pallas_call(
        body,
        grid=(np_ // bn,),
        in_specs=node_in + part_in + [
            blk((512, 512), lambda i: (0, 0)),
            blk((1, 512), lambda i: (0, 0)),
            blk((1024, 512), lambda i: (0, 0)),
            blk((1024, 1), lambda i: (0, 0)),
        ],
        out_specs=[blk((1024, bn), lambda i: (0, i))],
        out_shape=[jax.ShapeDtypeStruct((1024, np_), jnp.float32)],
    )(*chunks, *([a3] * 8), W3a, b3a, W3b, b3bcol)[0]


def kernel(x, emb, graph_data, e, Wg, bg, Wc, bc, W1a, b1a, W1b, b1b,
           W2a, b2a, W2b, b2b, W3a, b3a, W3b, b3b, Wfc, bfc):
    n = x.shape[2]
    ne = graph_data.shape[1]
    np_ = ((n + 1023) // 1024) * 1024
    npa = np_ + 128
    bn = 2048
    nwin = -(-(ne // _W) // (_NC * _NS * 8)) * (_NC * _NS * 8)
    wpt = nwin // (_NC * _NS)

    color = (e == 'color')

    xp = jnp.pad(x[0], ((0, 0), (0, np_ - n)))
    ep = jnp.pad(emb[0], ((0, 0), (0, np_ - n)))
    src, dst = graph_data[0], graph_data[1]
    pad_e = nwin * _W - ne
    ar = jnp.arange(pad_e, dtype=jnp.int32)
    src_w = jnp.concatenate([src, (ar * 37) % jnp.int32(n)]).reshape(nwin, _W)
    dst_w = jnp.concatenate([dst, np_ + (ar % 16)]).reshape(nwin, _W)
    z128 = jnp.zeros((npa, 128), jnp.float32)
    row = lambda b: b.reshape(1, -1)
    col = lambda b: b.reshape(-1, 1)

    f, oth0, oth1 = _stage_a(xp, ep, Wg, row(bg), Wc, row(bc), Wfc, row(bfc),
                             color, np_, bn)

    a1 = _seg_sum_partials([f], src_w, dst_w, z128, npa, 128, wpt)[0]
    (f1,), f1m = _gin_mlp(f, a1, W1a, row(b1a),
                          W1b, row(b1b), col(b1b), True, np_, bn)

    a2 = _seg_sum_partials([f1], src_w, dst_w, z128, npa, 128, wpt)[0]
    (f2a, f2b), f2m = _gin_mlp(f1, a2, W2a, row(b2a),
                               W2b, row(b2b), col(b2b), True, np_, bn)

    chunks = [f2a, f2b, oth0, oth1]
    a3 = _seg_sum_partials(chunks, src_w, dst_w, z128, npa, 128, wpt)
    f3m = _stage_d(chunks, a3, W3a, row(b3a), W3b, row(b3b), col(b3b),
                   np_, bn)

    out = jnp.concatenate([f1m, f2m, f3m], axis=0)[:, :n]
    return out[None]
